# baseline (speedup 1.0000x reference)
import jax
import jax.numpy as jnp
from jax.experimental import pallas as pl
from jax.experimental.pallas import tpu as pltpu

_B, _K, _D = 64, 1, 1024
_HS, _HR, _E = 4096, 1024, 8
_NS = 4
_CS = _HS // _NS
_G = _NS + _E


def _step(x_ref, rw_ref, wg_s_ref, bg_s_ref, wu_s_ref, bu_s_ref, wd_s_ref,
          bd_s_ref, wg_r_ref, bg_r_ref, wu_r_ref, bu_r_ref, wd_r_ref,
          bd_r_ref, out_ref):
    i = pl.program_id(0)
    xv = x_ref[...]

    @pl.when(i < _NS)
    def _shared():
        g = jnp.dot(xv, wg_s_ref[...], preferred_element_type=jnp.float32)
        u = jnp.dot(xv, wu_s_ref[...], preferred_element_type=jnp.float32)
        h = jax.nn.silu(g + bg_s_ref[...]) * (u + bu_s_ref[...])
        acc = jnp.dot(h, wd_s_ref[...], preferred_element_type=jnp.float32)

        @pl.when(i == 0)
        def _init():
            out_ref[...] = acc + bd_s_ref[...]

        @pl.when(i != 0)
        def _accum():
            out_ref[...] += acc

    @pl.when(i >= _NS)
    def _routed():
        w = rw_ref[0]
        g = jnp.dot(xv, wg_r_ref[0], preferred_element_type=jnp.float32)
        u = jnp.dot(xv, wu_r_ref[0], preferred_element_type=jnp.float32)
        h = jax.nn.silu(g + bg_r_ref[0]) * (u + bu_r_ref[0]) * w
        acc = jnp.dot(h, wd_r_ref[0], preferred_element_type=jnp.float32)
        out_ref[...] += acc + w * bd_r_ref[0]


def kernel(x, routing_weights, Wg_s, bg_s, Wu_s, bu_s, Wd_s, bd_s,
           Wg_r, bg_r, Wu_r, bu_r, Wd_r, bd_r):
    x2 = x.reshape(_B, _D)
    rw = routing_weights.T.reshape(_E, _B, 1)
    bg_r3 = bg_r.reshape(_E, 1, _HR)
    bu_r3 = bu_r.reshape(_E, 1, _HR)
    bd_r3 = bd_r.reshape(_E, 1, _D)

    def _sh(i):
        return jnp.minimum(i, _NS - 1)

    def _ex(i):
        return jnp.maximum(i - _NS, 0)

    out = pl.pallas_call(
        _step,
        grid=(_G,),
        in_specs=[
            pl.BlockSpec((_B, _D), lambda i: (0, 0)),
            pl.BlockSpec((1, _B, 1), lambda i: (_ex(i), 0, 0)),
            pl.BlockSpec((_D, _CS), lambda i: (0, _sh(i))),
            pl.BlockSpec((_CS,), lambda i: (_sh(i),)),
            pl.BlockSpec((_D, _CS), lambda i: (0, _sh(i))),
            pl.BlockSpec((_CS,), lambda i: (_sh(i),)),
            pl.BlockSpec((_CS, _D), lambda i: (_sh(i), 0)),
            pl.BlockSpec((_D,), lambda i: (0,)),
            pl.BlockSpec((1, _D, _HR), lambda i: (_ex(i), 0, 0)),
            pl.BlockSpec((1, 1, _HR), lambda i: (_ex(i), 0, 0)),
            pl.BlockSpec((1, _D, _HR), lambda i: (_ex(i), 0, 0)),
            pl.BlockSpec((1, 1, _HR), lambda i: (_ex(i), 0, 0)),
            pl.BlockSpec((1, _HR, _D), lambda i: (_ex(i), 0, 0)),
            pl.BlockSpec((1, 1, _D), lambda i: (_ex(i), 0, 0)),
        ],
        out_specs=pl.BlockSpec((_B, _D), lambda i: (0, 0)),
        out_shape=jax.ShapeDtypeStruct((_B, _D), jnp.float32),
        compiler_params=pltpu.CompilerParams(
            dimension_semantics=("arbitrary",),
        ),
    )(x2, rw, Wg_s, bg_s, Wu_s, bu_s, Wd_s, bd_s,
      Wg_r, bg_r3, Wu_r, bu_r3, Wd_r, bd_r3)

    return out.reshape(_B, _K, _D)
